# Initial kernel scaffold; baseline (speedup 1.0000x reference)
import functools

import jax
import jax.numpy as jnp
from jax import lax
from jax.experimental import pallas as pl
from jax.experimental.pallas import tpu as pltpu
from jax.experimental.pallas import tpu_sc as plsc

B = 4096
S = 200
D = 32
NC = 2
NS = 16
NW = NC * NS
TOTAL = B * S
PER_W = TOTAL // NW
R = 800
NCH = PER_W // R
REPS = R // S

_mesh = plsc.VectorSubcoreMesh(core_axis_name="c", subcore_axis_name="s")


@functools.partial(
    pl.kernel,
    mesh=_mesh,
    out_type=jax.ShapeDtypeStruct((TOTAL, D), jnp.float32),
    scratch_types=[
        pltpu.VMEM((R,), jnp.int32),
        pltpu.VMEM((R, D), jnp.float32),
        pltpu.VMEM((S, D), jnp.float32),
        pltpu.SemaphoreType.DMA,
    ],
)
def _embed(x_hbm, tok_hbm, pos_hbm, out_hbm, idx_v, rows_v, pos_v, sem):
    wid = lax.axis_index("s") * NC + lax.axis_index("c")
    base = wid * PER_W

    pltpu.sync_copy(pos_hbm, pos_v)

    for c in range(NCH):
        off = base + c * R
        pltpu.sync_copy(x_hbm.at[pl.ds(off, R)], idx_v)
        pltpu.async_copy(tok_hbm.at[idx_v], rows_v, sem).wait()

        def add_pos(p, _):
            lo = pos_v[p, pl.ds(0, 16)]
            hi = pos_v[p, pl.ds(16, 16)]
            for rep in range(REPS):
                r = rep * S + p
                rows_v[r, pl.ds(0, 16)] = rows_v[r, pl.ds(0, 16)] + lo
                rows_v[r, pl.ds(16, 16)] = rows_v[r, pl.ds(16, 16)] + hi
            return 0

        lax.fori_loop(0, S, add_pos, 0)

        pltpu.sync_copy(rows_v, out_hbm.at[pl.ds(off, R)])


def kernel(x, token_table, pos_table):
    xf = x.reshape(-1).astype(jnp.int32)
    out = _embed(xf, token_table, pos_table)
    return out.reshape(B, S, D)


# speedup vs baseline: 1.3923x; 1.3923x over previous
import functools

import jax
import jax.numpy as jnp
from jax import lax
from jax.experimental import pallas as pl
from jax.experimental.pallas import tpu as pltpu
from jax.experimental.pallas import tpu_sc as plsc

B = 4096
S = 200
D = 32
NC = 2
NS = 16
NW = NC * NS
TOTAL = B * S
PER_W = TOTAL // NW
R = 800
NCH = PER_W // R
REPS = R // S

_mesh = plsc.VectorSubcoreMesh(core_axis_name="c", subcore_axis_name="s")


@functools.partial(
    pl.kernel,
    mesh=_mesh,
    compiler_params=pltpu.CompilerParams(use_tc_tiling_on_sc=False),
    out_type=jax.ShapeDtypeStruct((TOTAL, D), jnp.float32),
    scratch_types=[
        pltpu.VMEM((R,), jnp.int32),
        pltpu.VMEM((R, D), jnp.float32),
        pltpu.VMEM((S, D), jnp.float32),
        pltpu.SemaphoreType.DMA,
    ],
)
def _embed(x_hbm, tok_hbm, pos_hbm, out_hbm, idx_v, rows_v, pos_v, sem):
    wid = lax.axis_index("s") * NC + lax.axis_index("c")
    base = wid * PER_W

    pltpu.sync_copy(pos_hbm, pos_v)

    for c in range(NCH):
        off = base + c * R
        pltpu.sync_copy(x_hbm.at[pl.ds(off, R)], idx_v)
        pltpu.async_copy(tok_hbm.at[idx_v], rows_v, sem).wait()

        def add_pos(p, _):
            lo = pos_v[p, pl.ds(0, 16)]
            hi = pos_v[p, pl.ds(16, 16)]
            for rep in range(REPS):
                r = rep * S + p
                rows_v[r, pl.ds(0, 16)] = rows_v[r, pl.ds(0, 16)] + lo
                rows_v[r, pl.ds(16, 16)] = rows_v[r, pl.ds(16, 16)] + hi
            return 0

        lax.fori_loop(0, S, add_pos, 0)

        pltpu.sync_copy(rows_v, out_hbm.at[pl.ds(off, R)])


def kernel(x, token_table, pos_table):
    xf = x.reshape(-1).astype(jnp.int32)
    out = _embed(xf, token_table, pos_table)
    return out.reshape(B, S, D)
